# Initial kernel scaffold; baseline (speedup 1.0000x reference)
import jax
import jax.numpy as jnp
from jax.experimental import pallas as pl


def kernel(inputs, emb, W, b):
    raise NotImplementedError("write your pallas kernel here")


# speedup vs baseline: 6.6192x; 6.6192x over previous
import functools

import jax
import jax.numpy as jnp
from jax import lax
from jax.experimental import pallas as pl
from jax.experimental.pallas import tpu as pltpu
from jax.experimental.pallas import tpu_sc as plsc

_VOCAB = 100000
_D = 64
_B = 4096
_L = 200

_NC = 2
_NS = 16
_NW = _NC * _NS
_RPW = _B // _NW

_VB = 2048


def _sc_bag_body(idx_hbm, emb_hbm, out_hbm, idx_v, buf_v, acc_v, sem):
    c = lax.axis_index("c")
    s = lax.axis_index("s")
    wid = s * _NC + c
    base = wid * _RPW

    pltpu.sync_copy(idx_hbm.at[pl.ds(base, _RPW)], idx_v)

    def gather_row(r, dst):
        d0 = pltpu.async_copy(emb_hbm.at[idx_v.at[r, 0]], dst.at[pl.ds(0, _L // 2)], sem)
        d1 = pltpu.async_copy(emb_hbm.at[idx_v.at[r, 1]], dst.at[pl.ds(_L // 2, _L // 2)], sem)
        d0.wait()
        d1.wait()

    gather_row(0, buf_v)

    def init_copy(l, _):
        for cc in range(_D // 16):
            sl = pl.ds(cc * 16, 16)
            acc_v[l, sl] = buf_v[l, sl]
        return 0

    lax.fori_loop(0, _L, init_copy, 0)

    def accumulate(l, _):
        for cc in range(_D // 16):
            sl = pl.ds(cc * 16, 16)
            plsc.addupdate(acc_v.at[l, sl], buf_v[l, sl])
        return 0

    def row_body(r, _):
        gather_row(r, buf_v)
        lax.fori_loop(0, _L, accumulate, 0)
        return 0

    lax.fori_loop(1, _RPW, row_body, 0)

    pltpu.sync_copy(acc_v, out_hbm.at[wid])


@functools.cache
def _sc_bag():
    return pl.kernel(
        _sc_bag_body,
        out_type=jax.ShapeDtypeStruct((_NW, _L, _D), jnp.float32),
        mesh=plsc.VectorSubcoreMesh(core_axis_name="c", subcore_axis_name="s"),
        scratch_types=[
            pltpu.VMEM((_RPW, 2, _L // 2), jnp.int32),
            pltpu.VMEM((_L, _D), jnp.float32),
            pltpu.VMEM((_L, _D), jnp.float32),
            pltpu.SemaphoreType.DMA,
        ],
        compiler_params=pltpu.CompilerParams(use_tc_tiling_on_sc=False),
    )


def _tc_proj_body(p_ref, w_ref, b_ref, o_ref, s_ref):
    @pl.when(pl.program_id(0) == 0)
    def _():
        s_ref[...] = jnp.sum(p_ref[...], axis=0) * (1.0 / _B)

    o_ref[...] = (
        lax.dot_general(
            s_ref[...],
            w_ref[...],
            (((1,), (1,)), ((), ())),
            preferred_element_type=jnp.float32,
        )
        + b_ref[...]
    )


def _tc_proj(partials, W, b2):
    grid = (pl.cdiv(_VOCAB, _VB),)
    return pl.pallas_call(
        _tc_proj_body,
        grid=grid,
        in_specs=[
            pl.BlockSpec((_NW, _L, _D), lambda i: (0, 0, 0)),
            pl.BlockSpec((_VB, _D), lambda i: (i, 0)),
            pl.BlockSpec((1, _VB), lambda i: (0, i)),
        ],
        out_specs=pl.BlockSpec((_L, _VB), lambda i: (0, i)),
        out_shape=jax.ShapeDtypeStruct((_L, _VOCAB), jnp.float32),
        scratch_shapes=[pltpu.VMEM((_L, _D), jnp.float32)],
    )(partials, W, b2)


@jax.jit
def kernel(inputs, emb, W, b):
    idx = inputs.astype(jnp.int32).reshape(_B, 2, _L // 2)
    partials = _sc_bag()(idx, emb)
    return _tc_proj(partials, W, b.reshape(1, _VOCAB))


# speedup vs baseline: 9.2106x; 1.3915x over previous
import functools

import jax
import jax.numpy as jnp
from jax import lax
from jax.experimental import pallas as pl
from jax.experimental.pallas import tpu as pltpu
from jax.experimental.pallas import tpu_sc as plsc

_VOCAB = 100000
_D = 64
_B = 4096
_L = 200

_NC = 2
_NS = 16
_NW = _NC * _NS
_RPW = _B // _NW

_VB = 2048


def _sc_bag_body(idx_hbm, emb_hbm, out_hbm, idx_v, buf0_v, buf1_v, acc_v, sem0, sem1):
    c = lax.axis_index("c")
    s = lax.axis_index("s")
    wid = s * _NC + c
    base = wid * _RPW
    half = _L // 2

    pltpu.sync_copy(idx_hbm.at[pl.ds(base, _RPW)], idx_v)

    def fire(r, buf, sem):
        pltpu.async_copy(emb_hbm.at[idx_v.at[r, 0]], buf.at[pl.ds(0, half)], sem)
        pltpu.async_copy(emb_hbm.at[idx_v.at[r, 1]], buf.at[pl.ds(half, half)], sem)

    def drain(buf, sem):
        pltpu.make_async_copy(emb_hbm.at[pl.ds(0, _L)], buf, sem).wait()

    def init(buf):
        @plsc.parallel_loop(0, _L, unroll=4)
        def _(l):
            for cc in range(_D // 16):
                sl = pl.ds(cc * 16, 16)
                acc_v[l, sl] = buf[l, sl]

    def accum(buf):
        @plsc.parallel_loop(0, _L, unroll=4)
        def _(l):
            for cc in range(_D // 16):
                sl = pl.ds(cc * 16, 16)
                plsc.addupdate(acc_v.at[l, sl], buf[l, sl])

    fire(0, buf0_v, sem0)
    fire(1, buf1_v, sem1)
    drain(buf0_v, sem0)
    init(buf0_v)
    fire(2, buf0_v, sem0)

    def body(g, _):
        drain(buf1_v, sem1)
        accum(buf1_v)
        fire(2 * g + 1, buf1_v, sem1)
        drain(buf0_v, sem0)
        accum(buf0_v)

        @pl.when(g < _RPW // 2 - 1)
        def _():
            fire(2 * g + 2, buf0_v, sem0)

        return 0

    lax.fori_loop(1, _RPW // 2, body, 0)

    drain(buf1_v, sem1)
    accum(buf1_v)

    pltpu.sync_copy(acc_v, out_hbm.at[wid])


@functools.cache
def _sc_bag():
    return pl.kernel(
        _sc_bag_body,
        out_type=jax.ShapeDtypeStruct((_NW, _L, _D), jnp.float32),
        mesh=plsc.VectorSubcoreMesh(core_axis_name="c", subcore_axis_name="s"),
        scratch_types=[
            pltpu.VMEM((_RPW, 2, _L // 2), jnp.int32),
            pltpu.VMEM((_L, _D), jnp.float32),
            pltpu.VMEM((_L, _D), jnp.float32),
            pltpu.VMEM((_L, _D), jnp.float32),
            pltpu.SemaphoreType.DMA,
            pltpu.SemaphoreType.DMA,
        ],
        compiler_params=pltpu.CompilerParams(use_tc_tiling_on_sc=False),
    )


def _tc_proj_body(p_ref, w_ref, b_ref, o_ref, s_ref):
    @pl.when(pl.program_id(0) == 0)
    def _():
        s_ref[...] = jnp.sum(p_ref[...], axis=0) * (1.0 / _B)

    o_ref[...] = (
        lax.dot_general(
            s_ref[...],
            w_ref[...],
            (((1,), (1,)), ((), ())),
            preferred_element_type=jnp.float32,
        )
        + b_ref[...]
    )


def _tc_proj(partials, W, b2):
    grid = (pl.cdiv(_VOCAB, _VB),)
    return pl.pallas_call(
        _tc_proj_body,
        grid=grid,
        in_specs=[
            pl.BlockSpec((_NW, _L, _D), lambda i: (0, 0, 0)),
            pl.BlockSpec((_VB, _D), lambda i: (i, 0)),
            pl.BlockSpec((1, _VB), lambda i: (0, i)),
        ],
        out_specs=pl.BlockSpec((_L, _VB), lambda i: (0, i)),
        out_shape=jax.ShapeDtypeStruct((_L, _VOCAB), jnp.float32),
        scratch_shapes=[pltpu.VMEM((_L, _D), jnp.float32)],
    )(partials, W, b2)


@jax.jit
def kernel(inputs, emb, W, b):
    idx = inputs.astype(jnp.int32).reshape(_B, 2, _L // 2)
    partials = _sc_bag()(idx, emb)
    return _tc_proj(partials, W, b.reshape(1, _VOCAB))


# speedup vs baseline: 10.1818x; 1.1054x over previous
import functools

import jax
import jax.numpy as jnp
from jax import lax
from jax.experimental import pallas as pl
from jax.experimental.pallas import tpu as pltpu
from jax.experimental.pallas import tpu_sc as plsc

_VOCAB = 100000
_D = 64
_B = 4096
_L = 200

_NC = 2
_NS = 16
_NW = _NC * _NS
_RPW = _B // _NW

_VB = 2048


def _sc_bag_body(
    idx_hbm, emb_hbm, out_hbm, idx_v, a0_v, a1_v, b0_v, b1_v, acc_v, sem0, sem1
):
    c = lax.axis_index("c")
    s = lax.axis_index("s")
    wid = s * _NC + c
    base = wid * _RPW
    half = _L // 2

    pltpu.sync_copy(idx_hbm.at[pl.ds(base, _RPW)], idx_v)

    def fire_pair(r, bufA, bufB, sem):
        pltpu.async_copy(emb_hbm.at[idx_v.at[r, 0]], bufA.at[pl.ds(0, half)], sem)
        pltpu.async_copy(emb_hbm.at[idx_v.at[r, 1]], bufA.at[pl.ds(half, half)], sem)
        pltpu.async_copy(emb_hbm.at[idx_v.at[r + 1, 0]], bufB.at[pl.ds(0, half)], sem)
        pltpu.async_copy(emb_hbm.at[idx_v.at[r + 1, 1]], bufB.at[pl.ds(half, half)], sem)

    def drain_pair(bufA, bufB, sem):
        pltpu.make_async_copy(emb_hbm.at[pl.ds(0, _L)], bufA, sem).wait()
        pltpu.make_async_copy(emb_hbm.at[pl.ds(0, _L)], bufB, sem).wait()

    def init2(bufA, bufB):
        @plsc.parallel_loop(0, _L, unroll=4)
        def _(l):
            for cc in range(_D // 16):
                sl = pl.ds(cc * 16, 16)
                acc_v[l, sl] = bufA[l, sl] + bufB[l, sl]

    def accum2(bufA, bufB):
        @plsc.parallel_loop(0, _L, unroll=4)
        def _(l):
            for cc in range(_D // 16):
                sl = pl.ds(cc * 16, 16)
                plsc.addupdate(acc_v.at[l, sl], bufA[l, sl] + bufB[l, sl])

    fire_pair(0, a0_v, a1_v, sem0)
    fire_pair(2, b0_v, b1_v, sem1)
    drain_pair(a0_v, a1_v, sem0)
    init2(a0_v, a1_v)
    fire_pair(4, a0_v, a1_v, sem0)

    def body(g, _):
        drain_pair(b0_v, b1_v, sem1)
        accum2(b0_v, b1_v)
        fire_pair(4 * g + 2, b0_v, b1_v, sem1)
        drain_pair(a0_v, a1_v, sem0)
        accum2(a0_v, a1_v)

        @pl.when(g < _RPW // 4 - 1)
        def _():
            fire_pair(4 * g + 4, a0_v, a1_v, sem0)

        return 0

    lax.fori_loop(1, _RPW // 4, body, 0)

    drain_pair(b0_v, b1_v, sem1)
    accum2(b0_v, b1_v)

    pltpu.sync_copy(acc_v, out_hbm.at[wid])


@functools.cache
def _sc_bag():
    return pl.kernel(
        _sc_bag_body,
        out_type=jax.ShapeDtypeStruct((_NW, _L, _D), jnp.float32),
        mesh=plsc.VectorSubcoreMesh(core_axis_name="c", subcore_axis_name="s"),
        scratch_types=[
            pltpu.VMEM((_RPW, 2, _L // 2), jnp.int32),
            pltpu.VMEM((_L, _D), jnp.float32),
            pltpu.VMEM((_L, _D), jnp.float32),
            pltpu.VMEM((_L, _D), jnp.float32),
            pltpu.VMEM((_L, _D), jnp.float32),
            pltpu.VMEM((_L, _D), jnp.float32),
            pltpu.SemaphoreType.DMA,
            pltpu.SemaphoreType.DMA,
        ],
        compiler_params=pltpu.CompilerParams(use_tc_tiling_on_sc=False),
    )


def _tc_proj_body(p_ref, w_ref, b_ref, o_ref, s_ref):
    @pl.when(pl.program_id(0) == 0)
    def _():
        s_ref[...] = jnp.sum(p_ref[...], axis=0) * (1.0 / _B)

    o_ref[...] = (
        lax.dot_general(
            s_ref[...],
            w_ref[...],
            (((1,), (1,)), ((), ())),
            preferred_element_type=jnp.float32,
        )
        + b_ref[...]
    )


def _tc_proj(partials, W, b2):
    grid = (pl.cdiv(_VOCAB, _VB),)
    return pl.pallas_call(
        _tc_proj_body,
        grid=grid,
        in_specs=[
            pl.BlockSpec((_NW, _L, _D), lambda i: (0, 0, 0)),
            pl.BlockSpec((_VB, _D), lambda i: (i, 0)),
            pl.BlockSpec((1, _VB), lambda i: (0, i)),
        ],
        out_specs=pl.BlockSpec((_L, _VB), lambda i: (0, i)),
        out_shape=jax.ShapeDtypeStruct((_L, _VOCAB), jnp.float32),
        scratch_shapes=[pltpu.VMEM((_L, _D), jnp.float32)],
    )(partials, W, b2)


@jax.jit
def kernel(inputs, emb, W, b):
    idx = inputs.astype(jnp.int32).reshape(_B, 2, _L // 2)
    partials = _sc_bag()(idx, emb)
    return _tc_proj(partials, W, b.reshape(1, _VOCAB))


# speedup vs baseline: 10.8703x; 1.0676x over previous
import functools

import jax
import jax.numpy as jnp
from jax import lax
from jax.experimental import pallas as pl
from jax.experimental.pallas import tpu as pltpu
from jax.experimental.pallas import tpu_sc as plsc

_VOCAB = 100000
_D = 64
_B = 4096
_L = 200

_NC = 2
_NS = 16
_NW = _NC * _NS
_RPW = _B // _NW

_VB = 4096


def _sc_bag_body(
    idx_hbm, emb_hbm, out_hbm, idx_v, a0_v, a1_v, b0_v, b1_v, acc_v, sem0, sem1
):
    c = lax.axis_index("c")
    s = lax.axis_index("s")
    wid = s * _NC + c
    base = wid * _RPW
    half = _L // 2

    pltpu.sync_copy(idx_hbm.at[pl.ds(base, _RPW)], idx_v)

    def fire_pair(r, bufA, bufB, sem):
        pltpu.async_copy(emb_hbm.at[idx_v.at[r, 0]], bufA.at[pl.ds(0, half)], sem)
        pltpu.async_copy(emb_hbm.at[idx_v.at[r, 1]], bufA.at[pl.ds(half, half)], sem)
        pltpu.async_copy(emb_hbm.at[idx_v.at[r + 1, 0]], bufB.at[pl.ds(0, half)], sem)
        pltpu.async_copy(emb_hbm.at[idx_v.at[r + 1, 1]], bufB.at[pl.ds(half, half)], sem)

    def drain_pair(bufA, bufB, sem):
        pltpu.make_async_copy(emb_hbm.at[pl.ds(0, _L)], bufA, sem).wait()
        pltpu.make_async_copy(emb_hbm.at[pl.ds(0, _L)], bufB, sem).wait()

    def init2(bufA, bufB):
        @plsc.parallel_loop(0, _L, unroll=4)
        def _(l):
            for cc in range(_D // 16):
                sl = pl.ds(cc * 16, 16)
                acc_v[l, sl] = bufA[l, sl] + bufB[l, sl]

    def accum2(bufA, bufB):
        @plsc.parallel_loop(0, _L, unroll=4)
        def _(l):
            for cc in range(_D // 16):
                sl = pl.ds(cc * 16, 16)
                plsc.addupdate(acc_v.at[l, sl], bufA[l, sl] + bufB[l, sl])

    fire_pair(0, a0_v, a1_v, sem0)
    fire_pair(2, b0_v, b1_v, sem1)
    drain_pair(a0_v, a1_v, sem0)
    init2(a0_v, a1_v)
    fire_pair(4, a0_v, a1_v, sem0)

    def body(g, _):
        drain_pair(b0_v, b1_v, sem1)
        accum2(b0_v, b1_v)
        fire_pair(4 * g + 2, b0_v, b1_v, sem1)
        drain_pair(a0_v, a1_v, sem0)
        accum2(a0_v, a1_v)

        @pl.when(g < _RPW // 4 - 1)
        def _():
            fire_pair(4 * g + 4, a0_v, a1_v, sem0)

        return 0

    lax.fori_loop(1, _RPW // 4, body, 0)

    drain_pair(b0_v, b1_v, sem1)
    accum2(b0_v, b1_v)

    pltpu.sync_copy(acc_v, out_hbm.at[wid])


@functools.cache
def _sc_bag():
    return pl.kernel(
        _sc_bag_body,
        out_type=jax.ShapeDtypeStruct((_NW, _L, _D), jnp.float32),
        mesh=plsc.VectorSubcoreMesh(core_axis_name="c", subcore_axis_name="s"),
        scratch_types=[
            pltpu.VMEM((_RPW, 2, _L // 2), jnp.int32),
            pltpu.VMEM((_L, _D), jnp.float32),
            pltpu.VMEM((_L, _D), jnp.float32),
            pltpu.VMEM((_L, _D), jnp.float32),
            pltpu.VMEM((_L, _D), jnp.float32),
            pltpu.VMEM((_L, _D), jnp.float32),
            pltpu.SemaphoreType.DMA,
            pltpu.SemaphoreType.DMA,
        ],
        compiler_params=pltpu.CompilerParams(use_tc_tiling_on_sc=False),
    )


def _tc_proj_body(p_ref, w_ref, b_ref, o_ref, s_ref):
    @pl.when(pl.program_id(0) == 0)
    def _():
        s_ref[...] = jnp.sum(p_ref[...], axis=0) * (1.0 / _B)

    o_ref[...] = (
        lax.dot_general(
            s_ref[...],
            w_ref[...],
            (((1,), (1,)), ((), ())),
            preferred_element_type=jnp.float32,
        )
        + b_ref[...]
    )


def _tc_proj(partials, W, b2):
    grid = (pl.cdiv(_VOCAB, _VB),)
    return pl.pallas_call(
        _tc_proj_body,
        grid=grid,
        in_specs=[
            pl.BlockSpec((_NW, _L, _D), lambda i: (0, 0, 0)),
            pl.BlockSpec((_VB, _D), lambda i: (i, 0)),
            pl.BlockSpec((1, _VB), lambda i: (0, i)),
        ],
        out_specs=pl.BlockSpec((_L, _VB), lambda i: (0, i)),
        out_shape=jax.ShapeDtypeStruct((_L, _VOCAB), jnp.float32),
        scratch_shapes=[pltpu.VMEM((_L, _D), jnp.float32)],
    )(partials, W, b2)


@jax.jit
def kernel(inputs, emb, W, b):
    idx = inputs.astype(jnp.int32).reshape(_B, 2, _L // 2)
    partials = _sc_bag()(idx, emb)
    return _tc_proj(partials, W, b.reshape(1, _VOCAB))


# speedup vs baseline: 11.0193x; 1.0137x over previous
import functools

import jax
import jax.numpy as jnp
from jax import lax
from jax.experimental import pallas as pl
from jax.experimental.pallas import tpu as pltpu
from jax.experimental.pallas import tpu_sc as plsc

_VOCAB = 100000
_D = 64
_B = 4096
_L = 200

_NC = 2
_NS = 16
_NW = _NC * _NS
_RPW = _B // _NW

_VB = 8192


def _sc_bag_body(
    idx_hbm, emb_hbm, out_hbm, idx_v, a0_v, a1_v, b0_v, b1_v, acc_v, sem0, sem1
):
    c = lax.axis_index("c")
    s = lax.axis_index("s")
    wid = s * _NC + c
    base = wid * _RPW
    half = _L // 2

    pltpu.sync_copy(idx_hbm.at[pl.ds(base, _RPW)], idx_v)

    def fire_pair(r, bufA, bufB, sem):
        pltpu.async_copy(emb_hbm.at[idx_v.at[r, 0]], bufA.at[pl.ds(0, half)], sem)
        pltpu.async_copy(emb_hbm.at[idx_v.at[r, 1]], bufA.at[pl.ds(half, half)], sem)
        pltpu.async_copy(emb_hbm.at[idx_v.at[r + 1, 0]], bufB.at[pl.ds(0, half)], sem)
        pltpu.async_copy(emb_hbm.at[idx_v.at[r + 1, 1]], bufB.at[pl.ds(half, half)], sem)

    def drain_pair(bufA, bufB, sem):
        pltpu.make_async_copy(emb_hbm.at[pl.ds(0, _L)], bufA, sem).wait()
        pltpu.make_async_copy(emb_hbm.at[pl.ds(0, _L)], bufB, sem).wait()

    def init2(bufA, bufB):
        @plsc.parallel_loop(0, _L, unroll=4)
        def _(l):
            for cc in range(_D // 16):
                sl = pl.ds(cc * 16, 16)
                acc_v[l, sl] = bufA[l, sl] + bufB[l, sl]

    def accum2(bufA, bufB):
        @plsc.parallel_loop(0, _L, unroll=4)
        def _(l):
            for cc in range(_D // 16):
                sl = pl.ds(cc * 16, 16)
                plsc.addupdate(acc_v.at[l, sl], bufA[l, sl] + bufB[l, sl])

    fire_pair(0, a0_v, a1_v, sem0)
    fire_pair(2, b0_v, b1_v, sem1)
    drain_pair(a0_v, a1_v, sem0)
    init2(a0_v, a1_v)
    fire_pair(4, a0_v, a1_v, sem0)

    def body(g, _):
        drain_pair(b0_v, b1_v, sem1)
        accum2(b0_v, b1_v)
        fire_pair(4 * g + 2, b0_v, b1_v, sem1)
        drain_pair(a0_v, a1_v, sem0)
        accum2(a0_v, a1_v)

        @pl.when(g < _RPW // 4 - 1)
        def _():
            fire_pair(4 * g + 4, a0_v, a1_v, sem0)

        return 0

    lax.fori_loop(1, _RPW // 4, body, 0)

    drain_pair(b0_v, b1_v, sem1)
    accum2(b0_v, b1_v)

    pltpu.sync_copy(acc_v, out_hbm.at[wid])


@functools.cache
def _sc_bag():
    return pl.kernel(
        _sc_bag_body,
        out_type=jax.ShapeDtypeStruct((_NW, _L, _D), jnp.float32),
        mesh=plsc.VectorSubcoreMesh(core_axis_name="c", subcore_axis_name="s"),
        scratch_types=[
            pltpu.VMEM((_RPW, 2, _L // 2), jnp.int32),
            pltpu.VMEM((_L, _D), jnp.float32),
            pltpu.VMEM((_L, _D), jnp.float32),
            pltpu.VMEM((_L, _D), jnp.float32),
            pltpu.VMEM((_L, _D), jnp.float32),
            pltpu.VMEM((_L, _D), jnp.float32),
            pltpu.SemaphoreType.DMA,
            pltpu.SemaphoreType.DMA,
        ],
        compiler_params=pltpu.CompilerParams(use_tc_tiling_on_sc=False),
    )


def _tc_proj_body(p_ref, w_ref, b_ref, o_ref, s_ref):
    @pl.when(pl.program_id(0) == 0)
    def _():
        s_ref[...] = jnp.sum(p_ref[...], axis=0) * (1.0 / _B)

    o_ref[...] = (
        lax.dot_general(
            s_ref[...],
            w_ref[...],
            (((1,), (1,)), ((), ())),
            preferred_element_type=jnp.float32,
        )
        + b_ref[...]
    )


def _tc_proj(partials, W, b2):
    grid = (pl.cdiv(_VOCAB, _VB),)
    return pl.pallas_call(
        _tc_proj_body,
        grid=grid,
        in_specs=[
            pl.BlockSpec((_NW, _L, _D), lambda i: (0, 0, 0)),
            pl.BlockSpec((_VB, _D), lambda i: (i, 0)),
            pl.BlockSpec((1, _VB), lambda i: (0, i)),
        ],
        out_specs=pl.BlockSpec((_L, _VB), lambda i: (0, i)),
        out_shape=jax.ShapeDtypeStruct((_L, _VOCAB), jnp.float32),
        scratch_shapes=[pltpu.VMEM((_L, _D), jnp.float32)],
    )(partials, W, b2)


@jax.jit
def kernel(inputs, emb, W, b):
    idx = inputs.astype(jnp.int32).reshape(_B, 2, _L // 2)
    partials = _sc_bag()(idx, emb)
    return _tc_proj(partials, W, b.reshape(1, _VOCAB))


# speedup vs baseline: 11.2092x; 1.0172x over previous
import functools

import jax
import jax.numpy as jnp
from jax import lax
from jax.experimental import pallas as pl
from jax.experimental.pallas import tpu as pltpu
from jax.experimental.pallas import tpu_sc as plsc

_VOCAB = 100000
_D = 64
_B = 4096
_L = 200

_NC = 2
_NS = 16
_NW = _NC * _NS
_RPW = _B // _NW

_VB = 8192


def _sc_bag_body(
    idx_hbm, emb_hbm, out_hbm,
    idx_v, a0_v, a1_v, a2_v, a3_v, b0_v, b1_v, b2_v, b3_v, acc_v, sem0, sem1,
):
    c = lax.axis_index("c")
    s = lax.axis_index("s")
    wid = s * _NC + c
    base = wid * _RPW
    half = _L // 2

    pltpu.sync_copy(idx_hbm.at[pl.ds(base, _RPW)], idx_v)

    bufs_a = (a0_v, a1_v, a2_v, a3_v)
    bufs_b = (b0_v, b1_v, b2_v, b3_v)

    def fire_quad(j, h, bufs, sem):
        for i in range(4):
            pltpu.async_copy(emb_hbm.at[idx_v.at[4 * j + i, h]], bufs[i], sem)

    def drain_quad(bufs, sem):
        for i in range(4):
            pltpu.make_async_copy(emb_hbm.at[pl.ds(0, half)], bufs[i], sem).wait()

    def init4(bufs, h):
        b0, b1, b2, b3 = bufs

        @plsc.parallel_loop(0, half, unroll=4)
        def _(l):
            for cc in range(_D // 16):
                sl = pl.ds(cc * 16, 16)
                acc_v[h * half + l, sl] = (b0[l, sl] + b1[l, sl]) + (
                    b2[l, sl] + b3[l, sl]
                )

    def accum4(bufs, h):
        b0, b1, b2, b3 = bufs

        @plsc.parallel_loop(0, half, unroll=4)
        def _(l):
            for cc in range(_D // 16):
                sl = pl.ds(cc * 16, 16)
                plsc.addupdate(
                    acc_v.at[h * half + l, sl],
                    (b0[l, sl] + b1[l, sl]) + (b2[l, sl] + b3[l, sl]),
                )

    nq = _RPW // 4

    fire_quad(0, 0, bufs_a, sem0)
    fire_quad(0, 1, bufs_b, sem1)
    drain_quad(bufs_a, sem0)
    init4(bufs_a, 0)
    fire_quad(1, 0, bufs_a, sem0)
    drain_quad(bufs_b, sem1)
    init4(bufs_b, 1)
    fire_quad(1, 1, bufs_b, sem1)

    def body(j, _):
        drain_quad(bufs_a, sem0)
        accum4(bufs_a, 0)

        @pl.when(j < nq - 1)
        def _():
            fire_quad(j + 1, 0, bufs_a, sem0)

        drain_quad(bufs_b, sem1)
        accum4(bufs_b, 1)

        @pl.when(j < nq - 1)
        def _():
            fire_quad(j + 1, 1, bufs_b, sem1)

        return 0

    lax.fori_loop(1, nq, body, 0)

    pltpu.sync_copy(acc_v, out_hbm.at[wid])


@functools.cache
def _sc_bag():
    return pl.kernel(
        _sc_bag_body,
        out_type=jax.ShapeDtypeStruct((_NW, _L, _D), jnp.float32),
        mesh=plsc.VectorSubcoreMesh(core_axis_name="c", subcore_axis_name="s"),
        scratch_types=[
            pltpu.VMEM((_RPW, 2, _L // 2), jnp.int32),
            pltpu.VMEM((_L // 2, _D), jnp.float32),
            pltpu.VMEM((_L // 2, _D), jnp.float32),
            pltpu.VMEM((_L // 2, _D), jnp.float32),
            pltpu.VMEM((_L // 2, _D), jnp.float32),
            pltpu.VMEM((_L // 2, _D), jnp.float32),
            pltpu.VMEM((_L // 2, _D), jnp.float32),
            pltpu.VMEM((_L // 2, _D), jnp.float32),
            pltpu.VMEM((_L // 2, _D), jnp.float32),
            pltpu.VMEM((_L, _D), jnp.float32),
            pltpu.SemaphoreType.DMA,
            pltpu.SemaphoreType.DMA,
        ],
        compiler_params=pltpu.CompilerParams(use_tc_tiling_on_sc=False),
    )


def _tc_proj_body(p_ref, w_ref, b_ref, o_ref, s_ref):
    @pl.when(pl.program_id(0) == 0)
    def _():
        s_ref[...] = jnp.sum(p_ref[...], axis=0) * (1.0 / _B)

    o_ref[...] = (
        lax.dot_general(
            s_ref[...],
            w_ref[...],
            (((1,), (1,)), ((), ())),
            preferred_element_type=jnp.float32,
        )
        + b_ref[...]
    )


def _tc_proj(partials, W, b2):
    grid = (pl.cdiv(_VOCAB, _VB),)
    return pl.pallas_call(
        _tc_proj_body,
        grid=grid,
        in_specs=[
            pl.BlockSpec((_NW, _L, _D), lambda i: (0, 0, 0)),
            pl.BlockSpec((_VB, _D), lambda i: (i, 0)),
            pl.BlockSpec((1, _VB), lambda i: (0, i)),
        ],
        out_specs=pl.BlockSpec((_L, _VB), lambda i: (0, i)),
        out_shape=jax.ShapeDtypeStruct((_L, _VOCAB), jnp.float32),
        scratch_shapes=[pltpu.VMEM((_L, _D), jnp.float32)],
    )(partials, W, b2)


@jax.jit
def kernel(inputs, emb, W, b):
    idx = inputs.astype(jnp.int32).reshape(_B, 2, _L // 2)
    partials = _sc_bag()(idx, emb)
    return _tc_proj(partials, W, b.reshape(1, _VOCAB))
